# baseline (speedup 1.0000x reference)
import functools

import jax
import jax.numpy as jnp
from jax import lax
from jax.experimental import pallas as pl
from jax.experimental.pallas import tpu as pltpu
from jax.experimental.pallas import tpu_sc as plsc

_B, _S = 4, 2048
_D = 2048
_DFF = 5632
_DSFF = 2816
_E = 8
_K = 2
_ZC = 0.001
_BC = 0.01
_N = _B * _S

_TBR = 512
_TBS = 512
_FS = 256
_TB = 512
_TBLOG = 9
_FFT = 512
_R = _N * _K + _E * _TB
_G = _R // _TB

_NC, _NS = 2, 16
_NW = _NC * _NS
_TOKW = _N // _NW
_CT = 16


def _router_body(x_ref, gw_ref, rmeta_ref, rw_ref, loss_ref, cf_ref, ci_ref,
                 zsum, wsum, csum):
    t = pl.program_id(0)
    nt = pl.num_programs(0)

    @pl.when(t == 0)
    def _init():
        zsum[0, 0] = 0.0
        wsum[...] = jnp.zeros_like(wsum)
        csum[...] = jnp.zeros_like(csum)

    x = x_ref[...]
    logits = lax.dot_general(x, gw_ref[...], (((1,), (1,)), ((), ())),
                             preferred_element_type=jnp.float32)
    m = jnp.max(logits, axis=1, keepdims=True)
    ex = jnp.exp(logits - m)
    s = jnp.sum(ex, axis=1, keepdims=True)
    lse = m + jnp.log(s)
    w = ex / s

    idx8 = lax.broadcasted_iota(jnp.int32, (_TBR, _E), 1)
    m0 = jnp.max(w, axis=1, keepdims=True)
    i0 = jnp.min(jnp.where(w == m0, idx8, _E), axis=1, keepdims=True)
    oh0 = idx8 == i0
    wm = jnp.where(oh0, -1.0, w)
    m1 = jnp.max(wm, axis=1, keepdims=True)
    i1 = jnp.min(jnp.where(wm == m1, idx8, _E), axis=1, keepdims=True)
    oh1 = idx8 == i1
    tot = m0 + m1
    a0 = m0 / tot
    a1 = m1 / tot

    oh01 = (oh0 | oh1).astype(jnp.float32)
    rows = lax.broadcasted_iota(jnp.int32, (_TBR, _TBR), 0)
    cols = lax.broadcasted_iota(jnp.int32, (_TBR, _TBR), 1)
    tril = (cols <= rows).astype(jnp.float32)
    inc = lax.dot_general(tril, oh01, (((1,), (0,)), ((), ())),
                          preferred_element_type=jnp.float32)
    prior = csum[...].astype(jnp.float32) + (inc - oh01)
    r0 = jnp.sum(jnp.where(oh0, prior, 0.0), axis=1,
                 keepdims=True).astype(jnp.int32)
    r1 = jnp.sum(jnp.where(oh1, prior, 0.0), axis=1,
                 keepdims=True).astype(jnp.int32)

    zero_i = jnp.zeros((_TBR, _E), jnp.int32)
    rmeta_ref[...] = (jnp.where(idx8 == 0, i0, zero_i)
                      + jnp.where(idx8 == 1, i1, zero_i)
                      + jnp.where(idx8 == 2, r0, zero_i)
                      + jnp.where(idx8 == 3, r1, zero_i))
    zero_f = jnp.zeros((_TBR, _E), jnp.float32)
    rw_ref[...] = (jnp.where(idx8 == 0, a0, zero_f)
                   + jnp.where(idx8 == 1, a1, zero_f))

    zsum[0, 0] += jnp.sum(lse * lse)
    wsum[...] += jnp.sum(w, axis=0, keepdims=True)
    csum[...] += jnp.sum(oh01, axis=0, keepdims=True).astype(jnp.int32)

    @pl.when(t == nt - 1)
    def _fin():
        cf = csum[...].astype(jnp.float32)
        cf_ref[...] = cf
        cnt = csum[...]
        padded = lax.shift_right_logical(cnt + (_TB - 1), _TBLOG) * _TB
        li = lax.broadcasted_iota(jnp.int32, (_E, _E), 0)
        lj = lax.broadcasted_iota(jnp.int32, (_E, _E), 1)
        mstrict = (li < lj).astype(jnp.float32)
        ps = lax.dot_general(padded.astype(jnp.float32), mstrict,
                             (((1,), (0,)), ((), ())),
                             preferred_element_type=jnp.float32)
        ci_ref[...] = jnp.concatenate(
            [cnt, ps.astype(jnp.int32)], axis=1)
        z_loss = _ZC * zsum[0, 0] / _N
        bal = _BC * _E * jnp.sum((wsum[...] / _N) * (cf / (_N * _K)))
        loss_ref[0, 0] = z_loss + bal


def _router(xf, gate_w):
    return pl.pallas_call(
        _router_body,
        grid=(_N // _TBR,),
        in_specs=[
            pl.BlockSpec((_TBR, _D), lambda t: (t, 0)),
            pl.BlockSpec((_E, _D), lambda t: (0, 0)),
        ],
        out_specs=[
            pl.BlockSpec((_TBR, _E), lambda t: (t, 0)),
            pl.BlockSpec((_TBR, _E), lambda t: (t, 0)),
            pl.BlockSpec(memory_space=pltpu.SMEM),
            pl.BlockSpec((1, _E), lambda t: (0, 0)),
            pl.BlockSpec((1, 2 * _E), lambda t: (0, 0)),
        ],
        out_shape=[
            jax.ShapeDtypeStruct((_N, _E), jnp.int32),
            jax.ShapeDtypeStruct((_N, _E), jnp.float32),
            jax.ShapeDtypeStruct((1, 1), jnp.float32),
            jax.ShapeDtypeStruct((1, _E), jnp.float32),
            jax.ShapeDtypeStruct((1, 2 * _E), jnp.int32),
        ],
        scratch_shapes=[
            pltpu.SMEM((1, 1), jnp.float32),
            pltpu.VMEM((1, _E), jnp.float32),
            pltpu.VMEM((1, _E), jnp.int32),
        ],
    )(xf, gate_w)


def _block_meta(ci):
    counts = ci.reshape(2 * _E)[:_E]
    padded = ((counts + _TB - 1) // _TB) * _TB
    ps = jnp.cumsum(padded) - padded
    n_active = jnp.sum(padded) // _TB
    g = jnp.arange(_G, dtype=jnp.int32)
    be_raw = jnp.sum((g[:, None] * _TB >= ps[None, :]).astype(jnp.int32),
                     axis=1) - 1
    last_be = jnp.take(be_raw, jnp.maximum(n_active - 1, 0))
    act = (g < n_active).astype(jnp.int32)
    be = jnp.where(act == 1, be_raw, last_be)
    return be, act


def _sc_dest(ci_v, meta_v, row, zero):
    e0 = plsc.load_gather(meta_v, [row, zero])
    e1 = plsc.load_gather(meta_v, [row, zero + 1])
    r0 = plsc.load_gather(meta_v, [row, zero + 2])
    r1 = plsc.load_gather(meta_v, [row, zero + 3])
    d0 = plsc.load_gather(ci_v, [e0 + _E]) + r0
    d1 = plsc.load_gather(ci_v, [e1 + _E]) + r1
    return d0, d1


def _dispatch_body(rmeta_hbm, ci_hbm, xf_hbm, xs_hbm,
                   ci_v, meta_v, i0_v, i1_v, rows_v, sem):
    wid = lax.axis_index("s") * _NC + lax.axis_index("c")
    base_t = wid * _TOKW
    pltpu.sync_copy(ci_hbm, ci_v)

    row = lax.iota(jnp.int32, _CT)
    zero = jnp.zeros((_CT,), jnp.int32)

    def chunk(i, carry):
        t0 = base_t + i * _CT
        pltpu.sync_copy(rmeta_hbm.at[pl.ds(t0, _CT)], meta_v)
        d0, d1 = _sc_dest(ci_v, meta_v, row, zero)
        i0_v[...] = d0
        i1_v[...] = d1
        pltpu.sync_copy(xf_hbm.at[pl.ds(t0, _CT)], rows_v)
        pltpu.async_copy(rows_v, xs_hbm.at[i0_v], sem).wait()
        pltpu.async_copy(rows_v, xs_hbm.at[i1_v], sem).wait()
        return carry

    lax.fori_loop(0, _TOKW // _CT, chunk, 0)


def _dispatch(rmeta, ci, xf):
    mesh = plsc.VectorSubcoreMesh(core_axis_name="c", subcore_axis_name="s")
    return pl.kernel(
        _dispatch_body,
        out_type=jax.ShapeDtypeStruct((_R, _D), jnp.float32),
        mesh=mesh,
        compiler_params=pltpu.CompilerParams(needs_layout_passes=False),
        scratch_types=[
            pltpu.VMEM((2 * _E,), jnp.int32),
            pltpu.VMEM((_CT, _E), jnp.int32),
            pltpu.VMEM((_CT,), jnp.int32),
            pltpu.VMEM((_CT,), jnp.int32),
            pltpu.VMEM((_CT, _D), jnp.float32),
            pltpu.SemaphoreType.DMA,
        ],
    )(rmeta, ci, xf)


def _group_body(be_ref, act_ref, xs_ref, w1_ref, w3_ref, w2_ref, out_ref):
    g = pl.program_id(0)
    f = pl.program_id(1)

    @pl.when(f == 0)
    def _init():
        out_ref[...] = jnp.zeros_like(out_ref)

    @pl.when(act_ref[g] == 1)
    def _compute():
        x = xs_ref[...].astype(jnp.bfloat16)
        a = lax.dot_general(x, w1_ref[0], (((1,), (1,)), ((), ())),
                            preferred_element_type=jnp.float32)
        b = lax.dot_general(x, w3_ref[0], (((1,), (1,)), ((), ())),
                            preferred_element_type=jnp.float32)
        h = ((a * jax.nn.sigmoid(a)) * b).astype(jnp.bfloat16)
        out_ref[...] += lax.dot_general(h, w2_ref[0], (((1,), (1,)), ((), ())),
                                        preferred_element_type=jnp.float32)


def _grouped_mlp(be, act, xs, w1, w2, w3):
    grid_spec = pltpu.PrefetchScalarGridSpec(
        num_scalar_prefetch=2,
        grid=(_G, _DFF // _FFT),
        in_specs=[
            pl.BlockSpec((_TB, _D), lambda g, f, be, act: (g, 0)),
            pl.BlockSpec((1, _FFT, _D), lambda g, f, be, act: (be[g], f, 0)),
            pl.BlockSpec((1, _FFT, _D), lambda g, f, be, act: (be[g], f, 0)),
            pl.BlockSpec((1, _D, _FFT), lambda g, f, be, act: (be[g], 0, f)),
        ],
        out_specs=pl.BlockSpec((_TB, _D), lambda g, f, be, act: (g, 0)),
    )
    return pl.pallas_call(
        _group_body,
        grid_spec=grid_spec,
        out_shape=jax.ShapeDtypeStruct((_R, _D), jnp.float32),
    )(be, act, xs, w1, w3, w2)


def _shared_body(x_ref, sw1_ref, sw3_ref, sw2_ref, out_ref):
    f = pl.program_id(1)

    @pl.when(f == 0)
    def _init():
        out_ref[...] = jnp.zeros_like(out_ref)

    x = x_ref[...].astype(jnp.bfloat16)
    a = lax.dot_general(x, sw1_ref[...], (((1,), (1,)), ((), ())),
                        preferred_element_type=jnp.float32)
    b = lax.dot_general(x, sw3_ref[...], (((1,), (1,)), ((), ())),
                        preferred_element_type=jnp.float32)
    h = ((a * jax.nn.sigmoid(a)) * b).astype(jnp.bfloat16)
    out_ref[...] += lax.dot_general(h, sw2_ref[...], (((1,), (1,)), ((), ())),
                                    preferred_element_type=jnp.float32)


def _shared(xf, sw1, sw2, sw3):
    return pl.pallas_call(
        _shared_body,
        grid=(_N // _TBS, _DSFF // _FS),
        in_specs=[
            pl.BlockSpec((_TBS, _D), lambda t, f: (t, 0)),
            pl.BlockSpec((_FS, _D), lambda t, f: (f, 0)),
            pl.BlockSpec((_FS, _D), lambda t, f: (f, 0)),
            pl.BlockSpec((_D, _FS), lambda t, f: (0, f)),
        ],
        out_specs=pl.BlockSpec((_TBS, _D), lambda t, f: (t, 0)),
        out_shape=jax.ShapeDtypeStruct((_N, _D), jnp.float32),
    )(xf, sw1, sw3, sw2)


def _combine_body(ys_hbm, sh_hbm, rmeta_hbm, rw_hbm, ci_hbm, out_hbm,
                  ci_v, meta_v, w_v, i0_v, i1_v, y0_v, y1_v, ob_v, sem):
    wid = lax.axis_index("s") * _NC + lax.axis_index("c")
    base_t = wid * _TOKW
    pltpu.sync_copy(ci_hbm, ci_v)

    row = lax.iota(jnp.int32, _CT)
    zero = jnp.zeros((_CT,), jnp.int32)

    def chunk(i, carry):
        t0 = base_t + i * _CT
        pltpu.sync_copy(rmeta_hbm.at[pl.ds(t0, _CT)], meta_v)
        pltpu.sync_copy(rw_hbm.at[pl.ds(t0, _CT)], w_v)
        d0, d1 = _sc_dest(ci_v, meta_v, row, zero)
        i0_v[...] = d0
        i1_v[...] = d1
        pltpu.async_copy(ys_hbm.at[i0_v], y0_v, sem).wait()
        pltpu.async_copy(ys_hbm.at[i1_v], y1_v, sem).wait()
        pltpu.sync_copy(sh_hbm.at[pl.ds(t0, _CT)], ob_v)
        a0v = plsc.load_gather(w_v, [row, zero])
        a1v = plsc.load_gather(w_v, [row, zero + 1])
        for tok in range(_CT):
            a0 = jnp.broadcast_to(a0v[tok], (16,))
            a1 = jnp.broadcast_to(a1v[tok], (16,))

            def col(j, c, tok=tok, a0=a0, a1=a1):
                sl = pl.ds(j * 16, 16)
                plsc.addupdate(ob_v.at[tok, sl],
                               y0_v[tok, sl] * a0 + y1_v[tok, sl] * a1)
                return c

            lax.fori_loop(0, _D // 16, col, 0)
        pltpu.sync_copy(ob_v, out_hbm.at[pl.ds(t0, _CT)])
        return carry

    lax.fori_loop(0, _TOKW // _CT, chunk, 0)


def _combine(ys, shared_out, rmeta, rw, ci):
    mesh = plsc.VectorSubcoreMesh(core_axis_name="c", subcore_axis_name="s")
    return pl.kernel(
        _combine_body,
        out_type=jax.ShapeDtypeStruct((_N, _D), jnp.float32),
        mesh=mesh,
        compiler_params=pltpu.CompilerParams(needs_layout_passes=False),
        scratch_types=[
            pltpu.VMEM((2 * _E,), jnp.int32),
            pltpu.VMEM((_CT, _E), jnp.int32),
            pltpu.VMEM((_CT, _E), jnp.float32),
            pltpu.VMEM((_CT,), jnp.int32),
            pltpu.VMEM((_CT,), jnp.int32),
            pltpu.VMEM((_CT, _D), jnp.float32),
            pltpu.VMEM((_CT, _D), jnp.float32),
            pltpu.VMEM((_CT, _D), jnp.float32),
            pltpu.SemaphoreType.DMA,
        ],
    )(ys, shared_out, rmeta, rw, ci)


def kernel(x, gate_w, w1, w2, w3, sw1, sw2, sw3):
    b, s, d = x.shape
    xf = x.reshape(-1, d)
    bf = jnp.bfloat16
    rmeta, rw, loss, cf, ci = _router(xf, gate_w)
    ci_flat = ci.reshape(2 * _E)
    be, act = _block_meta(ci_flat)
    xs = _dispatch(rmeta, ci_flat, xf)
    ys = _grouped_mlp(be, act, xs, w1.astype(bf), w2.astype(bf), w3.astype(bf))
    shared_out = _shared(xf, sw1.astype(bf), sw2.astype(bf), sw3.astype(bf))
    out = _combine(ys, shared_out, rmeta, rw, ci_flat)
    return (out.reshape(b, s, d), loss.reshape(()), cf.reshape(_E))


# speedup vs baseline: 1.0925x; 1.0925x over previous
import functools

import jax
import jax.numpy as jnp
from jax import lax
from jax.experimental import pallas as pl
from jax.experimental.pallas import tpu as pltpu
from jax.experimental.pallas import tpu_sc as plsc

_B, _S = 4, 2048
_D = 2048
_DFF = 5632
_DSFF = 2816
_E = 8
_K = 2
_ZC = 0.001
_BC = 0.01
_N = _B * _S

_TBR = 512
_TBS = 512
_FS = 256
_TB = 1024
_TBLOG = 10
_FFT = 256
_R = _N * _K + _E * _TB
_G = _R // _TB

_NC, _NS = 2, 16
_NW = _NC * _NS
_TOKW = _N // _NW
_CT = 16


def _router_body(x_ref, gw_ref, rmeta_ref, rw_ref, loss_ref, cf_ref, ci_ref,
                 zsum, wsum, csum):
    t = pl.program_id(0)
    nt = pl.num_programs(0)

    @pl.when(t == 0)
    def _init():
        zsum[0, 0] = 0.0
        wsum[...] = jnp.zeros_like(wsum)
        csum[...] = jnp.zeros_like(csum)

    x = x_ref[...]
    logits = lax.dot_general(x, gw_ref[...], (((1,), (1,)), ((), ())),
                             preferred_element_type=jnp.float32)
    m = jnp.max(logits, axis=1, keepdims=True)
    ex = jnp.exp(logits - m)
    s = jnp.sum(ex, axis=1, keepdims=True)
    lse = m + jnp.log(s)
    w = ex / s

    idx8 = lax.broadcasted_iota(jnp.int32, (_TBR, _E), 1)
    m0 = jnp.max(w, axis=1, keepdims=True)
    i0 = jnp.min(jnp.where(w == m0, idx8, _E), axis=1, keepdims=True)
    oh0 = idx8 == i0
    wm = jnp.where(oh0, -1.0, w)
    m1 = jnp.max(wm, axis=1, keepdims=True)
    i1 = jnp.min(jnp.where(wm == m1, idx8, _E), axis=1, keepdims=True)
    oh1 = idx8 == i1
    tot = m0 + m1
    a0 = m0 / tot
    a1 = m1 / tot

    oh01 = (oh0 | oh1).astype(jnp.float32)
    rows = lax.broadcasted_iota(jnp.int32, (_TBR, _TBR), 0)
    cols = lax.broadcasted_iota(jnp.int32, (_TBR, _TBR), 1)
    tril = (cols <= rows).astype(jnp.float32)
    inc = lax.dot_general(tril, oh01, (((1,), (0,)), ((), ())),
                          preferred_element_type=jnp.float32)
    prior = csum[...].astype(jnp.float32) + (inc - oh01)
    r0 = jnp.sum(jnp.where(oh0, prior, 0.0), axis=1,
                 keepdims=True).astype(jnp.int32)
    r1 = jnp.sum(jnp.where(oh1, prior, 0.0), axis=1,
                 keepdims=True).astype(jnp.int32)

    zero_i = jnp.zeros((_TBR, _E), jnp.int32)
    rmeta_ref[...] = (jnp.where(idx8 == 0, i0, zero_i)
                      + jnp.where(idx8 == 1, i1, zero_i)
                      + jnp.where(idx8 == 2, r0, zero_i)
                      + jnp.where(idx8 == 3, r1, zero_i))
    zero_f = jnp.zeros((_TBR, _E), jnp.float32)
    rw_ref[...] = (jnp.where(idx8 == 0, a0, zero_f)
                   + jnp.where(idx8 == 1, a1, zero_f))

    zsum[0, 0] += jnp.sum(lse * lse)
    wsum[...] += jnp.sum(w, axis=0, keepdims=True)
    csum[...] += jnp.sum(oh01, axis=0, keepdims=True).astype(jnp.int32)

    @pl.when(t == nt - 1)
    def _fin():
        cf = csum[...].astype(jnp.float32)
        cf_ref[...] = cf
        cnt = csum[...]
        padded = lax.shift_right_logical(cnt + (_TB - 1), _TBLOG) * _TB
        li = lax.broadcasted_iota(jnp.int32, (_E, _E), 0)
        lj = lax.broadcasted_iota(jnp.int32, (_E, _E), 1)
        mstrict = (li < lj).astype(jnp.float32)
        ps = lax.dot_general(padded.astype(jnp.float32), mstrict,
                             (((1,), (0,)), ((), ())),
                             preferred_element_type=jnp.float32)
        ci_ref[...] = jnp.concatenate(
            [cnt, ps.astype(jnp.int32)], axis=1)
        z_loss = _ZC * zsum[0, 0] / _N
        bal = _BC * _E * jnp.sum((wsum[...] / _N) * (cf / (_N * _K)))
        loss_ref[0, 0] = z_loss + bal


def _router(xf, gate_w):
    return pl.pallas_call(
        _router_body,
        grid=(_N // _TBR,),
        in_specs=[
            pl.BlockSpec((_TBR, _D), lambda t: (t, 0)),
            pl.BlockSpec((_E, _D), lambda t: (0, 0)),
        ],
        out_specs=[
            pl.BlockSpec((_TBR, _E), lambda t: (t, 0)),
            pl.BlockSpec((_TBR, _E), lambda t: (t, 0)),
            pl.BlockSpec(memory_space=pltpu.SMEM),
            pl.BlockSpec((1, _E), lambda t: (0, 0)),
            pl.BlockSpec((1, 2 * _E), lambda t: (0, 0)),
        ],
        out_shape=[
            jax.ShapeDtypeStruct((_N, _E), jnp.int32),
            jax.ShapeDtypeStruct((_N, _E), jnp.float32),
            jax.ShapeDtypeStruct((1, 1), jnp.float32),
            jax.ShapeDtypeStruct((1, _E), jnp.float32),
            jax.ShapeDtypeStruct((1, 2 * _E), jnp.int32),
        ],
        scratch_shapes=[
            pltpu.SMEM((1, 1), jnp.float32),
            pltpu.VMEM((1, _E), jnp.float32),
            pltpu.VMEM((1, _E), jnp.int32),
        ],
    )(xf, gate_w)


def _block_meta(ci):
    counts = ci.reshape(2 * _E)[:_E]
    padded = ((counts + _TB - 1) // _TB) * _TB
    ps = jnp.cumsum(padded) - padded
    n_active = jnp.sum(padded) // _TB
    g = jnp.arange(_G, dtype=jnp.int32)
    be_raw = jnp.sum((g[:, None] * _TB >= ps[None, :]).astype(jnp.int32),
                     axis=1) - 1
    last_be = jnp.take(be_raw, jnp.maximum(n_active - 1, 0))
    act = (g < n_active).astype(jnp.int32)
    be = jnp.where(act == 1, be_raw, last_be)
    return be, act


def _sc_dest(ci_v, meta_v, row, zero):
    e0 = plsc.load_gather(meta_v, [row, zero])
    e1 = plsc.load_gather(meta_v, [row, zero + 1])
    r0 = plsc.load_gather(meta_v, [row, zero + 2])
    r1 = plsc.load_gather(meta_v, [row, zero + 3])
    d0 = plsc.load_gather(ci_v, [e0 + _E]) + r0
    d1 = plsc.load_gather(ci_v, [e1 + _E]) + r1
    return d0, d1


def _dispatch_body(rmeta_hbm, ci_hbm, xf_hbm, xs_hbm,
                   ci_v, meta_v, i0_v, i1_v, rows_v, sem):
    wid = lax.axis_index("s") * _NC + lax.axis_index("c")
    base_t = wid * _TOKW
    pltpu.sync_copy(ci_hbm, ci_v)

    row = lax.iota(jnp.int32, _CT)
    zero = jnp.zeros((_CT,), jnp.int32)

    def chunk(i, carry):
        t0 = base_t + i * _CT
        pltpu.sync_copy(rmeta_hbm.at[pl.ds(t0, _CT)], meta_v)
        d0, d1 = _sc_dest(ci_v, meta_v, row, zero)
        i0_v[...] = d0
        i1_v[...] = d1
        pltpu.sync_copy(xf_hbm.at[pl.ds(t0, _CT)], rows_v)
        pltpu.async_copy(rows_v, xs_hbm.at[i0_v], sem).wait()
        pltpu.async_copy(rows_v, xs_hbm.at[i1_v], sem).wait()
        return carry

    lax.fori_loop(0, _TOKW // _CT, chunk, 0)


def _dispatch(rmeta, ci, xf):
    mesh = plsc.VectorSubcoreMesh(core_axis_name="c", subcore_axis_name="s")
    return pl.kernel(
        _dispatch_body,
        out_type=jax.ShapeDtypeStruct((_R, _D), jnp.float32),
        mesh=mesh,
        compiler_params=pltpu.CompilerParams(needs_layout_passes=False),
        scratch_types=[
            pltpu.VMEM((2 * _E,), jnp.int32),
            pltpu.VMEM((_CT, _E), jnp.int32),
            pltpu.VMEM((_CT,), jnp.int32),
            pltpu.VMEM((_CT,), jnp.int32),
            pltpu.VMEM((_CT, _D), jnp.float32),
            pltpu.SemaphoreType.DMA,
        ],
    )(rmeta, ci, xf)


def _group_body(be_ref, act_ref, xs_ref, w1_ref, w3_ref, w2_ref, out_ref):
    g = pl.program_id(0)
    f = pl.program_id(1)

    @pl.when(f == 0)
    def _init():
        out_ref[...] = jnp.zeros_like(out_ref)

    @pl.when(act_ref[g] == 1)
    def _compute():
        x = xs_ref[...]
        a = lax.dot_general(x, w1_ref[0], (((1,), (1,)), ((), ())),
                            preferred_element_type=jnp.float32)
        b = lax.dot_general(x, w3_ref[0], (((1,), (1,)), ((), ())),
                            preferred_element_type=jnp.float32)
        h = (a * jax.nn.sigmoid(a)) * b
        out_ref[...] += lax.dot_general(h, w2_ref[0], (((1,), (1,)), ((), ())),
                                        preferred_element_type=jnp.float32)


def _grouped_mlp(be, act, xs, w1, w2, w3):
    grid_spec = pltpu.PrefetchScalarGridSpec(
        num_scalar_prefetch=2,
        grid=(_G, _DFF // _FFT),
        in_specs=[
            pl.BlockSpec((_TB, _D), lambda g, f, be, act: (g, 0)),
            pl.BlockSpec((1, _FFT, _D), lambda g, f, be, act: (be[g], f, 0)),
            pl.BlockSpec((1, _FFT, _D), lambda g, f, be, act: (be[g], f, 0)),
            pl.BlockSpec((1, _D, _FFT), lambda g, f, be, act: (be[g], 0, f)),
        ],
        out_specs=pl.BlockSpec((_TB, _D), lambda g, f, be, act: (g, 0)),
    )
    return pl.pallas_call(
        _group_body,
        grid_spec=grid_spec,
        out_shape=jax.ShapeDtypeStruct((_R, _D), jnp.float32),
    )(be, act, xs, w1, w3, w2)


def _shared_body(x_ref, sw1_ref, sw3_ref, sw2_ref, out_ref):
    f = pl.program_id(1)

    @pl.when(f == 0)
    def _init():
        out_ref[...] = jnp.zeros_like(out_ref)

    x = x_ref[...]
    a = lax.dot_general(x, sw1_ref[...], (((1,), (1,)), ((), ())),
                        preferred_element_type=jnp.float32)
    b = lax.dot_general(x, sw3_ref[...], (((1,), (1,)), ((), ())),
                        preferred_element_type=jnp.float32)
    h = (a * jax.nn.sigmoid(a)) * b
    out_ref[...] += lax.dot_general(h, sw2_ref[...], (((1,), (1,)), ((), ())),
                                    preferred_element_type=jnp.float32)


def _shared(xf, sw1, sw2, sw3):
    return pl.pallas_call(
        _shared_body,
        grid=(_N // _TBS, _DSFF // _FS),
        in_specs=[
            pl.BlockSpec((_TBS, _D), lambda t, f: (t, 0)),
            pl.BlockSpec((_FS, _D), lambda t, f: (f, 0)),
            pl.BlockSpec((_FS, _D), lambda t, f: (f, 0)),
            pl.BlockSpec((_D, _FS), lambda t, f: (0, f)),
        ],
        out_specs=pl.BlockSpec((_TBS, _D), lambda t, f: (t, 0)),
        out_shape=jax.ShapeDtypeStruct((_N, _D), jnp.float32),
    )(xf, sw1, sw3, sw2)


def _combine_body(ys_hbm, sh_hbm, rmeta_hbm, rw_hbm, ci_hbm, out_hbm,
                  ci_v, meta_v, w_v, i0_v, i1_v, y0_v, y1_v, ob_v, sem):
    wid = lax.axis_index("s") * _NC + lax.axis_index("c")
    base_t = wid * _TOKW
    pltpu.sync_copy(ci_hbm, ci_v)

    row = lax.iota(jnp.int32, _CT)
    zero = jnp.zeros((_CT,), jnp.int32)

    def chunk(i, carry):
        t0 = base_t + i * _CT
        pltpu.sync_copy(rmeta_hbm.at[pl.ds(t0, _CT)], meta_v)
        pltpu.sync_copy(rw_hbm.at[pl.ds(t0, _CT)], w_v)
        d0, d1 = _sc_dest(ci_v, meta_v, row, zero)
        i0_v[...] = d0
        i1_v[...] = d1
        pltpu.async_copy(ys_hbm.at[i0_v], y0_v, sem).wait()
        pltpu.async_copy(ys_hbm.at[i1_v], y1_v, sem).wait()
        pltpu.sync_copy(sh_hbm.at[pl.ds(t0, _CT)], ob_v)
        a0v = plsc.load_gather(w_v, [row, zero])
        a1v = plsc.load_gather(w_v, [row, zero + 1])
        for tok in range(_CT):
            a0 = jnp.broadcast_to(a0v[tok], (16,))
            a1 = jnp.broadcast_to(a1v[tok], (16,))

            def col(j, c, tok=tok, a0=a0, a1=a1):
                sl = pl.ds(j * 16, 16)
                plsc.addupdate(ob_v.at[tok, sl],
                               y0_v[tok, sl] * a0 + y1_v[tok, sl] * a1)
                return c

            lax.fori_loop(0, _D // 16, col, 0)
        pltpu.sync_copy(ob_v, out_hbm.at[pl.ds(t0, _CT)])
        return carry

    lax.fori_loop(0, _TOKW // _CT, chunk, 0)


def _combine(ys, shared_out, rmeta, rw, ci):
    mesh = plsc.VectorSubcoreMesh(core_axis_name="c", subcore_axis_name="s")
    return pl.kernel(
        _combine_body,
        out_type=jax.ShapeDtypeStruct((_N, _D), jnp.float32),
        mesh=mesh,
        compiler_params=pltpu.CompilerParams(needs_layout_passes=False),
        scratch_types=[
            pltpu.VMEM((2 * _E,), jnp.int32),
            pltpu.VMEM((_CT, _E), jnp.int32),
            pltpu.VMEM((_CT, _E), jnp.float32),
            pltpu.VMEM((_CT,), jnp.int32),
            pltpu.VMEM((_CT,), jnp.int32),
            pltpu.VMEM((_CT, _D), jnp.float32),
            pltpu.VMEM((_CT, _D), jnp.float32),
            pltpu.VMEM((_CT, _D), jnp.float32),
            pltpu.SemaphoreType.DMA,
        ],
    )(ys, shared_out, rmeta, rw, ci)


def kernel(x, gate_w, w1, w2, w3, sw1, sw2, sw3):
    b, s, d = x.shape
    xf = x.reshape(-1, d)
    rmeta, rw, loss, cf, ci = _router(xf, gate_w)
    ci_flat = ci.reshape(2 * _E)
    be, act = _block_meta(ci_flat)
    xs = _dispatch(rmeta, ci_flat, xf)
    ys = _grouped_mlp(be, act, xs, w1, w2, w3)
    shared_out = _shared(xf, sw1, sw2, sw3)
    out = _combine(ys, shared_out, rmeta, rw, ci_flat)
    return (out.reshape(b, s, d), loss.reshape(()), cf.reshape(_E))


# speedup vs baseline: 1.0977x; 1.0047x over previous
import functools

import jax
import jax.numpy as jnp
from jax import lax
from jax.experimental import pallas as pl
from jax.experimental.pallas import tpu as pltpu
from jax.experimental.pallas import tpu_sc as plsc

_B, _S = 4, 2048
_D = 2048
_DFF = 5632
_DSFF = 2816
_E = 8
_K = 2
_ZC = 0.001
_BC = 0.01
_N = _B * _S

_TBR = 512
_TBS = 512
_FS = 256
_TB = 1024
_TBLOG = 10
_FFT = 256
_R = _N * _K + _E * _TB
_G = _R // _TB

_NC, _NS = 2, 16
_NW = _NC * _NS
_TOKW = _N // _NW
_CT = 16


def _router_body(x_ref, gw_ref, rmeta_ref, rw_ref, loss_ref, cf_ref, ci_ref,
                 zsum, wsum, csum):
    t = pl.program_id(0)
    nt = pl.num_programs(0)

    @pl.when(t == 0)
    def _init():
        zsum[0, 0] = 0.0
        wsum[...] = jnp.zeros_like(wsum)
        csum[...] = jnp.zeros_like(csum)

    x = x_ref[...]
    logits = lax.dot_general(x, gw_ref[...], (((1,), (1,)), ((), ())),
                             preferred_element_type=jnp.float32)
    m = jnp.max(logits, axis=1, keepdims=True)
    ex = jnp.exp(logits - m)
    s = jnp.sum(ex, axis=1, keepdims=True)
    lse = m + jnp.log(s)
    w = ex / s

    idx8 = lax.broadcasted_iota(jnp.int32, (_TBR, _E), 1)
    m0 = jnp.max(w, axis=1, keepdims=True)
    i0 = jnp.min(jnp.where(w == m0, idx8, _E), axis=1, keepdims=True)
    oh0 = idx8 == i0
    wm = jnp.where(oh0, -1.0, w)
    m1 = jnp.max(wm, axis=1, keepdims=True)
    i1 = jnp.min(jnp.where(wm == m1, idx8, _E), axis=1, keepdims=True)
    oh1 = idx8 == i1
    tot = m0 + m1
    a0 = m0 / tot
    a1 = m1 / tot

    oh01 = (oh0 | oh1).astype(jnp.float32)
    rows = lax.broadcasted_iota(jnp.int32, (_TBR, _TBR), 0)
    cols = lax.broadcasted_iota(jnp.int32, (_TBR, _TBR), 1)
    tril = (cols <= rows).astype(jnp.float32)
    inc = lax.dot_general(tril, oh01, (((1,), (0,)), ((), ())),
                          preferred_element_type=jnp.float32)
    prior = csum[...].astype(jnp.float32) + (inc - oh01)
    r0 = jnp.sum(jnp.where(oh0, prior, 0.0), axis=1,
                 keepdims=True).astype(jnp.int32)
    r1 = jnp.sum(jnp.where(oh1, prior, 0.0), axis=1,
                 keepdims=True).astype(jnp.int32)

    zero_i = jnp.zeros((_TBR, _E), jnp.int32)
    rmeta_ref[...] = (jnp.where(idx8 == 0, i0, zero_i)
                      + jnp.where(idx8 == 1, i1, zero_i)
                      + jnp.where(idx8 == 2, r0, zero_i)
                      + jnp.where(idx8 == 3, r1, zero_i))
    zero_f = jnp.zeros((_TBR, _E), jnp.float32)
    rw_ref[...] = (jnp.where(idx8 == 0, a0, zero_f)
                   + jnp.where(idx8 == 1, a1, zero_f))

    zsum[0, 0] += jnp.sum(lse * lse)
    wsum[...] += jnp.sum(w, axis=0, keepdims=True)
    csum[...] += jnp.sum(oh01, axis=0, keepdims=True).astype(jnp.int32)

    @pl.when(t == nt - 1)
    def _fin():
        cf = csum[...].astype(jnp.float32)
        cf_ref[...] = cf
        cnt = csum[...]
        padded = lax.shift_right_logical(cnt + (_TB - 1), _TBLOG) * _TB
        li = lax.broadcasted_iota(jnp.int32, (_E, _E), 0)
        lj = lax.broadcasted_iota(jnp.int32, (_E, _E), 1)
        mstrict = (li < lj).astype(jnp.float32)
        ps = lax.dot_general(padded.astype(jnp.float32), mstrict,
                             (((1,), (0,)), ((), ())),
                             preferred_element_type=jnp.float32)
        ci_ref[...] = jnp.concatenate(
            [cnt, ps.astype(jnp.int32)], axis=1)
        z_loss = _ZC * zsum[0, 0] / _N
        bal = _BC * _E * jnp.sum((wsum[...] / _N) * (cf / (_N * _K)))
        loss_ref[0, 0] = z_loss + bal


def _router(xf, gate_w):
    return pl.pallas_call(
        _router_body,
        grid=(_N // _TBR,),
        in_specs=[
            pl.BlockSpec((_TBR, _D), lambda t: (t, 0)),
            pl.BlockSpec((_E, _D), lambda t: (0, 0)),
        ],
        out_specs=[
            pl.BlockSpec((_TBR, _E), lambda t: (t, 0)),
            pl.BlockSpec((_TBR, _E), lambda t: (t, 0)),
            pl.BlockSpec(memory_space=pltpu.SMEM),
            pl.BlockSpec((1, _E), lambda t: (0, 0)),
            pl.BlockSpec((1, 2 * _E), lambda t: (0, 0)),
        ],
        out_shape=[
            jax.ShapeDtypeStruct((_N, _E), jnp.int32),
            jax.ShapeDtypeStruct((_N, _E), jnp.float32),
            jax.ShapeDtypeStruct((1, 1), jnp.float32),
            jax.ShapeDtypeStruct((1, _E), jnp.float32),
            jax.ShapeDtypeStruct((1, 2 * _E), jnp.int32),
        ],
        scratch_shapes=[
            pltpu.SMEM((1, 1), jnp.float32),
            pltpu.VMEM((1, _E), jnp.float32),
            pltpu.VMEM((1, _E), jnp.int32),
        ],
    )(xf, gate_w)


def _block_meta(ci):
    counts = ci.reshape(2 * _E)[:_E]
    padded = ((counts + _TB - 1) // _TB) * _TB
    ps = jnp.cumsum(padded) - padded
    n_active = jnp.sum(padded) // _TB
    g = jnp.arange(_G, dtype=jnp.int32)
    be_raw = jnp.sum((g[:, None] * _TB >= ps[None, :]).astype(jnp.int32),
                     axis=1) - 1
    last_be = jnp.take(be_raw, jnp.maximum(n_active - 1, 0))
    act = (g < n_active).astype(jnp.int32)
    be = jnp.where(act == 1, be_raw, last_be)
    return be, act


def _sc_dest(ci_v, meta_v, row, zero):
    e0 = plsc.load_gather(meta_v, [row, zero])
    e1 = plsc.load_gather(meta_v, [row, zero + 1])
    r0 = plsc.load_gather(meta_v, [row, zero + 2])
    r1 = plsc.load_gather(meta_v, [row, zero + 3])
    d0 = plsc.load_gather(ci_v, [e0 + _E]) + r0
    d1 = plsc.load_gather(ci_v, [e1 + _E]) + r1
    return d0, d1


def _dispatch_body(rmeta_hbm, ci_hbm, xf_hbm, xs_hbm,
                   ci_v, meta_v, i0_v, i1_v, rows_v, sem):
    wid = lax.axis_index("s") * _NC + lax.axis_index("c")
    base_t = wid * _TOKW
    pltpu.sync_copy(ci_hbm, ci_v)

    row = lax.iota(jnp.int32, _CT)
    zero = jnp.zeros((_CT,), jnp.int32)

    def chunk(i, carry):
        t0 = base_t + i * _CT
        pltpu.sync_copy(rmeta_hbm.at[pl.ds(t0, _CT)], meta_v)
        d0, d1 = _sc_dest(ci_v, meta_v, row, zero)
        i0_v[...] = d0
        i1_v[...] = d1
        pltpu.sync_copy(xf_hbm.at[pl.ds(t0, _CT)], rows_v)
        c0 = pltpu.async_copy(rows_v, xs_hbm.at[i0_v], sem)
        c1 = pltpu.async_copy(rows_v, xs_hbm.at[i1_v], sem)
        c0.wait()
        c1.wait()
        return carry

    lax.fori_loop(0, _TOKW // _CT, chunk, 0)


def _dispatch(rmeta, ci, xf):
    mesh = plsc.VectorSubcoreMesh(core_axis_name="c", subcore_axis_name="s")
    return pl.kernel(
        _dispatch_body,
        out_type=jax.ShapeDtypeStruct((_R, _D), jnp.float32),
        mesh=mesh,
        compiler_params=pltpu.CompilerParams(needs_layout_passes=False),
        scratch_types=[
            pltpu.VMEM((2 * _E,), jnp.int32),
            pltpu.VMEM((_CT, _E), jnp.int32),
            pltpu.VMEM((_CT,), jnp.int32),
            pltpu.VMEM((_CT,), jnp.int32),
            pltpu.VMEM((_CT, _D), jnp.float32),
            pltpu.SemaphoreType.DMA,
        ],
    )(rmeta, ci, xf)


def _group_body(be_ref, act_ref, xs_ref, w1_ref, w3_ref, w2_ref, out_ref):
    g = pl.program_id(0)
    f = pl.program_id(1)

    @pl.when(f == 0)
    def _init():
        out_ref[...] = jnp.zeros_like(out_ref)

    @pl.when(act_ref[g] == 1)
    def _compute():
        x = xs_ref[...]
        a = lax.dot_general(x, w1_ref[0], (((1,), (1,)), ((), ())),
                            preferred_element_type=jnp.float32)
        b = lax.dot_general(x, w3_ref[0], (((1,), (1,)), ((), ())),
                            preferred_element_type=jnp.float32)
        h = (a * jax.nn.sigmoid(a)) * b
        out_ref[...] += lax.dot_general(h, w2_ref[0], (((1,), (1,)), ((), ())),
                                        preferred_element_type=jnp.float32)


def _grouped_mlp(be, act, xs, w1, w2, w3):
    grid_spec = pltpu.PrefetchScalarGridSpec(
        num_scalar_prefetch=2,
        grid=(_G, _DFF // _FFT),
        in_specs=[
            pl.BlockSpec((_TB, _D), lambda g, f, be, act: (g, 0)),
            pl.BlockSpec((1, _FFT, _D), lambda g, f, be, act: (be[g], f, 0)),
            pl.BlockSpec((1, _FFT, _D), lambda g, f, be, act: (be[g], f, 0)),
            pl.BlockSpec((1, _D, _FFT), lambda g, f, be, act: (be[g], 0, f)),
        ],
        out_specs=pl.BlockSpec((_TB, _D), lambda g, f, be, act: (g, 0)),
    )
    return pl.pallas_call(
        _group_body,
        grid_spec=grid_spec,
        out_shape=jax.ShapeDtypeStruct((_R, _D), jnp.float32),
    )(be, act, xs, w1, w3, w2)


def _shared_body(x_ref, sw1_ref, sw3_ref, sw2_ref, out_ref):
    f = pl.program_id(1)

    @pl.when(f == 0)
    def _init():
        out_ref[...] = jnp.zeros_like(out_ref)

    x = x_ref[...]
    a = lax.dot_general(x, sw1_ref[...], (((1,), (1,)), ((), ())),
                        preferred_element_type=jnp.float32)
    b = lax.dot_general(x, sw3_ref[...], (((1,), (1,)), ((), ())),
                        preferred_element_type=jnp.float32)
    h = (a * jax.nn.sigmoid(a)) * b
    out_ref[...] += lax.dot_general(h, sw2_ref[...], (((1,), (1,)), ((), ())),
                                    preferred_element_type=jnp.float32)


def _shared(xf, sw1, sw2, sw3):
    return pl.pallas_call(
        _shared_body,
        grid=(_N // _TBS, _DSFF // _FS),
        in_specs=[
            pl.BlockSpec((_TBS, _D), lambda t, f: (t, 0)),
            pl.BlockSpec((_FS, _D), lambda t, f: (f, 0)),
            pl.BlockSpec((_FS, _D), lambda t, f: (f, 0)),
            pl.BlockSpec((_D, _FS), lambda t, f: (0, f)),
        ],
        out_specs=pl.BlockSpec((_TBS, _D), lambda t, f: (t, 0)),
        out_shape=jax.ShapeDtypeStruct((_N, _D), jnp.float32),
    )(xf, sw1, sw3, sw2)


def _combine_body(ys_hbm, sh_hbm, rmeta_hbm, rw_hbm, ci_hbm, out_hbm,
                  ci_v, meta_v, w_v, i0_v, i1_v, y0_v, y1_v, ob_v, sem):
    wid = lax.axis_index("s") * _NC + lax.axis_index("c")
    base_t = wid * _TOKW
    pltpu.sync_copy(ci_hbm, ci_v)

    row = lax.iota(jnp.int32, _CT)
    zero = jnp.zeros((_CT,), jnp.int32)

    def chunk(i, carry):
        t0 = base_t + i * _CT
        pltpu.sync_copy(rmeta_hbm.at[pl.ds(t0, _CT)], meta_v)
        pltpu.sync_copy(rw_hbm.at[pl.ds(t0, _CT)], w_v)
        d0, d1 = _sc_dest(ci_v, meta_v, row, zero)
        i0_v[...] = d0
        i1_v[...] = d1
        c0 = pltpu.async_copy(ys_hbm.at[i0_v], y0_v, sem)
        c1 = pltpu.async_copy(ys_hbm.at[i1_v], y1_v, sem)
        pltpu.sync_copy(sh_hbm.at[pl.ds(t0, _CT)], ob_v)
        c0.wait()
        c1.wait()
        a0v = plsc.load_gather(w_v, [row, zero])
        a1v = plsc.load_gather(w_v, [row, zero + 1])
        for tok in range(_CT):
            a0 = jnp.broadcast_to(a0v[tok], (16,))
            a1 = jnp.broadcast_to(a1v[tok], (16,))

            def col(j, c, tok=tok, a0=a0, a1=a1):
                sl = pl.ds(j * 16, 16)
                plsc.addupdate(ob_v.at[tok, sl],
                               y0_v[tok, sl] * a0 + y1_v[tok, sl] * a1)
                return c

            lax.fori_loop(0, _D // 16, col, 0)
        pltpu.sync_copy(ob_v, out_hbm.at[pl.ds(t0, _CT)])
        return carry

    lax.fori_loop(0, _TOKW // _CT, chunk, 0)


def _combine(ys, shared_out, rmeta, rw, ci):
    mesh = plsc.VectorSubcoreMesh(core_axis_name="c", subcore_axis_name="s")
    return pl.kernel(
        _combine_body,
        out_type=jax.ShapeDtypeStruct((_N, _D), jnp.float32),
        mesh=mesh,
        compiler_params=pltpu.CompilerParams(needs_layout_passes=False),
        scratch_types=[
            pltpu.VMEM((2 * _E,), jnp.int32),
            pltpu.VMEM((_CT, _E), jnp.int32),
            pltpu.VMEM((_CT, _E), jnp.float32),
            pltpu.VMEM((_CT,), jnp.int32),
            pltpu.VMEM((_CT,), jnp.int32),
            pltpu.VMEM((_CT, _D), jnp.float32),
            pltpu.VMEM((_CT, _D), jnp.float32),
            pltpu.VMEM((_CT, _D), jnp.float32),
            pltpu.SemaphoreType.DMA,
        ],
    )(ys, shared_out, rmeta, rw, ci)


def kernel(x, gate_w, w1, w2, w3, sw1, sw2, sw3):
    b, s, d = x.shape
    xf = x.reshape(-1, d)
    rmeta, rw, loss, cf, ci = _router(xf, gate_w)
    ci_flat = ci.reshape(2 * _E)
    be, act = _block_meta(ci_flat)
    xs = _dispatch(rmeta, ci_flat, xf)
    ys = _grouped_mlp(be, act, xs, w1, w2, w3)
    shared_out = _shared(xf, sw1, sw2, sw3)
    out = _combine(ys, shared_out, rmeta, rw, ci_flat)
    return (out.reshape(b, s, d), loss.reshape(()), cf.reshape(_E))
